# baseline (speedup 1.0000x reference)
import functools

import jax
import jax.numpy as jnp
from jax import lax
from jax.experimental import pallas as pl
from jax.experimental.pallas import tpu as pltpu
from jax.experimental.pallas import tpu_sc as plsc

N = 10000
D = 128
E = 320000
EPS = 1e-5

NC = 2
NS = 16
NW = NC * NS
CH = 64
NCH = 160
Q = 4
EPW = CH * NCH
E_PAD = NW * EPW
NCH_F = 200
NCH_S = 120
EPW_F = CH * NCH_F
EPW_S = CH * NCH_S
OFF_F = NS * EPW_S
E_PAD_E = OFF_F + NS * EPW_F
ACC_ROWS = 10112
DUMMY = 10000
ZS = ACC_ROWS // NS


@functools.cache
def _sc_kernels():
    mesh = plsc.VectorSubcoreMesh(core_axis_name="c", subcore_axis_name="s",
                                  num_cores=NC, num_subcores=NS)

    @functools.partial(
        pl.kernel,
        mesh=mesh,
        out_type=jax.ShapeDtypeStruct((NC, ACC_ROWS, D), jnp.float32),
        scratch_types=[
            pltpu.VMEM((NCH, CH), jnp.int32),
            pltpu.VMEM((CH, D), jnp.float32),
            pltpu.VMEM_SHARED((ACC_ROWS, D), jnp.float32),
        ],
    )
    def _sc_degree(dst_hbm, ones_hbm, zeros_hbm, out_hbm, dst_v, ones_v, acc):
        c = lax.axis_index("c")
        s = lax.axis_index("s")
        wid = c * NS + s
        pltpu.sync_copy(dst_hbm.at[wid], dst_v)
        pltpu.sync_copy(ones_hbm, ones_v)
        pltpu.sync_copy(zeros_hbm.at[pl.ds(s * ZS, ZS)],
                        acc.at[pl.ds(s * ZS, ZS)])
        plsc.subcore_barrier()

        def body(j, carry):
            pltpu.sync_copy(ones_v, acc.at[dst_v.at[j]], add=True)
            return carry

        lax.fori_loop(0, NCH, body, 0)
        plsc.subcore_barrier()
        pltpu.sync_copy(acc.at[pl.ds(s * ZS, ZS)],
                        out_hbm.at[c, pl.ds(s * ZS, ZS)])

    @functools.partial(
        pl.kernel,
        mesh=mesh,
        out_type=jax.ShapeDtypeStruct((NC, ACC_ROWS, D), jnp.float32),
        scratch_types=[
            pltpu.VMEM((Q, CH), jnp.int32),
            pltpu.VMEM((Q, CH), jnp.int32),
            pltpu.VMEM((Q * CH, D), jnp.float32),
            pltpu.VMEM_SHARED((ACC_ROWS, D), jnp.float32),
            [pltpu.SemaphoreType.DMA] * Q,
            [pltpu.SemaphoreType.DMA] * Q,
            [pltpu.SemaphoreType.DMA] * Q,
        ],
    )
    def _sc_edge_pass(h_hbm, src_hbm, dst_hbm, zeros_hbm, out_hbm,
                      sring, dring, rows, acc, sem_g, sem_is, sem_id):
        c = lax.axis_index("c")
        s = lax.axis_index("s")
        wid = c * NS + s
        pltpu.sync_copy(zeros_hbm.at[pl.ds(s * ZS, ZS)],
                        acc.at[pl.ds(s * ZS, ZS)])
        plsc.subcore_barrier()

        fast = c == 0
        nch = jnp.where(fast, NCH_F, NCH_S)
        base = pl.multiple_of(
            jnp.where(fast, OFF_F + s * EPW_F, s * EPW_S), CH)

        def src_chunk(j):
            return src_hbm.at[pl.ds(base + j * CH, CH)]

        def dst_chunk(j):
            return dst_hbm.at[pl.ds(base + j * CH, CH)]

        def slot(q):
            return rows.at[pl.ds(q * CH, CH)]

        for q in range(Q):
            pltpu.async_copy(src_chunk(q), sring.at[q], sem_is[q])
            pltpu.async_copy(dst_chunk(q), dring.at[q], sem_id[q])
        for q in range(Q):
            pltpu.make_async_copy(src_chunk(q), sring.at[q],
                                  sem_is[q]).wait()
            pltpu.async_copy(h_hbm.at[sring.at[q]], slot(q), sem_g[q])

        def body(m, carry):
            for q in range(Q):
                k = Q * m + q
                pltpu.make_async_copy(h_hbm.at[sring.at[q]], slot(q),
                                      sem_g[q]).wait()

                @pl.when(k + Q < nch)
                def _(q=q, k=k):
                    pltpu.async_copy(src_chunk(k + Q), sring.at[q],
                                     sem_is[q])

                pltpu.make_async_copy(dst_chunk(0), dring.at[q],
                                      sem_id[q]).wait()
                pltpu.sync_copy(slot(q), acc.at[dring.at[q]], add=True)

                @pl.when(k + Q < nch)
                def _(q=q, k=k):
                    pltpu.async_copy(dst_chunk(k + Q), dring.at[q],
                                     sem_id[q])
                    pltpu.make_async_copy(src_chunk(0), sring.at[q],
                                          sem_is[q]).wait()
                    pltpu.async_copy(h_hbm.at[sring.at[q]], slot(q),
                                     sem_g[q])

            return carry

        lax.fori_loop(0, nch // Q, body, 0)
        plsc.subcore_barrier()
        pltpu.sync_copy(acc.at[pl.ds(s * ZS, ZS)],
                        out_hbm.at[c, pl.ds(s * ZS, ZS)])

    return _sc_degree, _sc_edge_pass


def _tc_prep_body(x_ref, w_ref, d0_ref, d1_ref, dinv_ref, hp_ref):
    deg = d0_ref[...][:N, :1] + d1_ref[...][:N, :1] + 1.0
    dinv = lax.rsqrt(deg)
    dinv_ref[...] = dinv
    h = jnp.dot(x_ref[...], w_ref[...], preferred_element_type=jnp.float32)
    hp_ref[...] = h * dinv


def _tc_epi_body(relu, skip, wnext, refs):
    if skip and wnext:
        (s0, s1, hp, dinv, b, g, be, hprev, skw, w2, feat_ref, hn_ref) = refs
    elif wnext:
        (s0, s1, hp, dinv, b, g, be, w2, feat_ref, hn_ref) = refs
    else:
        (s0, s1, hp, dinv, b, g, be, feat_ref) = refs
    dv = dinv[...]
    conv = (s0[...][:N] + s1[...][:N] + hp[...]) * dv + b[...]
    m = jnp.mean(conv, axis=0, keepdims=True)
    v = jnp.mean((conv - m) ** 2, axis=0, keepdims=True)
    h = (conv - m) * lax.rsqrt(v + EPS) * g[...] + be[...]
    if relu:
        h = jnp.maximum(h, 0.0)
    if skip:
        h = h + skw[...] * hprev[...]
    feat_ref[...] = h
    if wnext:
        hn_ref[...] = jnp.dot(h, w2[...],
                              preferred_element_type=jnp.float32) * dv


_f = jnp.float32


def _tc_prep(x, w, d0, d1):
    return pl.pallas_call(
        _tc_prep_body,
        out_shape=(jax.ShapeDtypeStruct((N, 1), _f),
                   jax.ShapeDtypeStruct((N, D), _f)),
    )(x, w, d0, d1)


def _tc_epi(relu, skip, wnext, *args):
    outs = [jax.ShapeDtypeStruct((N, D), _f)]
    if wnext:
        outs.append(jax.ShapeDtypeStruct((N, D), _f))
    body = functools.partial(_tc_epi_body, relu, skip, wnext)
    return pl.pallas_call(
        lambda *refs: body(refs),
        out_shape=tuple(outs) if wnext else outs[0],
    )(*args)


def kernel(x, edge_index, W0, b0, g0, be0, W1, b1, g1, be1,
           W2, b2, g2, be2, W3, b3, g3, be3, skip_weight):
    src = edge_index[0].astype(jnp.int32)
    dst = edge_index[1].astype(jnp.int32)

    pad = E_PAD - E
    dst3 = jnp.concatenate([dst, jnp.full((pad,), DUMMY, jnp.int32)]
                           ).reshape(NW, NCH, CH)

    fper = (E - OFF_F) // NS
    fpad = EPW_F - fper

    def _split(a, fill):
        fast_part = a[OFF_F:].reshape(NS, fper)
        fast_part = jnp.pad(fast_part, ((0, 0), (0, fpad)),
                            constant_values=fill)
        return jnp.concatenate([a[:OFF_F], fast_part.reshape(-1)])

    src1 = _split(src, 0)
    dst1 = _split(dst, DUMMY)
    zeros_h = jnp.zeros((ACC_ROWS, D), _f)
    ones_h = jnp.ones((CH, D), _f)

    _sc_degree, _sc_edge_pass = _sc_kernels()
    dpar = _sc_degree(dst3, ones_h, zeros_h)
    dinv, hp = _tc_prep(x, W0, dpar[0], dpar[1])

    def edge(hprime):
        S = _sc_edge_pass(hprime, src1, dst1, zeros_h)
        return S[0], S[1]

    b0r, g0r, be0r = b0.reshape(1, D), g0.reshape(1, D), be0.reshape(1, D)
    b1r, g1r, be1r = b1.reshape(1, D), g1.reshape(1, D), be1.reshape(1, D)
    b2r, g2r, be2r = b2.reshape(1, D), g2.reshape(1, D), be2.reshape(1, D)
    b3r, g3r, be3r = b3.reshape(1, D), g3.reshape(1, D), be3.reshape(1, D)
    skw = jnp.asarray(skip_weight, _f).reshape(1, 1)

    s0, s1 = edge(hp)
    feat0, hp1 = _tc_epi(True, False, True, s0, s1, hp, dinv,
                         b0r, g0r, be0r, W1)
    s0, s1 = edge(hp1)
    feat1, hp2 = _tc_epi(True, True, True, s0, s1, hp1, dinv,
                         b1r, g1r, be1r, feat0, skw, W2)
    s0, s1 = edge(hp2)
    feat2, hp3 = _tc_epi(True, True, True, s0, s1, hp2, dinv,
                         b2r, g2r, be2r, feat1, skw, W3)
    s0, s1 = edge(hp3)
    out = _tc_epi(False, False, False, s0, s1, hp3, dinv, b3r, g3r, be3r)
    return out


# speedup vs baseline: 1.5496x; 1.5496x over previous
import functools

import jax
import jax.numpy as jnp
from jax import lax
from jax.experimental import pallas as pl
from jax.experimental.pallas import tpu as pltpu
from jax.experimental.pallas import tpu_sc as plsc

N = 10000
D = 128
E = 320000
EPS = 1e-5

NC = 2
NS = 16
NW = NC * NS
CH = 64
NCH = 160
Q = 4
EPW = CH * NCH
E_PAD = NW * EPW
NCH_F = 188
NCH_S = 128
EPW_F = CH * NCH_F
EPW_S = CH * NCH_S
OFF_F = NS * EPW_S
E_PAD_E = OFF_F + NS * EPW_F
ACC_ROWS = 10112
DUMMY = 10000
ZS = ACC_ROWS // NS


@functools.cache
def _sc_kernels():
    mesh = plsc.VectorSubcoreMesh(core_axis_name="c", subcore_axis_name="s",
                                  num_cores=NC, num_subcores=NS)

    @functools.partial(
        pl.kernel,
        mesh=mesh,
        out_type=jax.ShapeDtypeStruct((NC, ACC_ROWS, D), jnp.float32),
        scratch_types=[
            pltpu.VMEM((NCH, CH), jnp.int32),
            pltpu.VMEM((CH, D), jnp.float32),
            pltpu.VMEM_SHARED((ACC_ROWS, D), jnp.float32),
        ],
    )
    def _sc_degree(dst_hbm, ones_hbm, zeros_hbm, out_hbm, dst_v, ones_v, acc):
        c = lax.axis_index("c")
        s = lax.axis_index("s")
        wid = c * NS + s
        pltpu.sync_copy(dst_hbm.at[wid], dst_v)
        pltpu.sync_copy(ones_hbm, ones_v)
        pltpu.sync_copy(zeros_hbm.at[pl.ds(s * ZS, ZS)],
                        acc.at[pl.ds(s * ZS, ZS)])
        plsc.subcore_barrier()

        def body(j, carry):
            pltpu.sync_copy(ones_v, acc.at[dst_v.at[j]], add=True)
            return carry

        lax.fori_loop(0, NCH, body, 0)
        plsc.subcore_barrier()
        pltpu.sync_copy(acc.at[pl.ds(s * ZS, ZS)],
                        out_hbm.at[c, pl.ds(s * ZS, ZS)])

    @functools.partial(
        pl.kernel,
        mesh=mesh,
        out_type=jax.ShapeDtypeStruct((NC, ACC_ROWS, D), jnp.float32),
        scratch_types=[
            pltpu.VMEM((Q, CH), jnp.int32),
            pltpu.VMEM((Q, CH), jnp.int32),
            pltpu.VMEM((Q * CH, D), jnp.float32),
            pltpu.VMEM_SHARED((ACC_ROWS, D), jnp.float32),
            [pltpu.SemaphoreType.DMA] * Q,
            [pltpu.SemaphoreType.DMA] * Q,
            [pltpu.SemaphoreType.DMA] * Q,
        ],
    )
    def _sc_edge_pass(h_hbm, src_hbm, dst_hbm, zeros_hbm, out_hbm,
                      sring, dring, rows, acc, sem_g, sem_is, sem_id):
        c = lax.axis_index("c")
        s = lax.axis_index("s")
        wid = c * NS + s
        pltpu.sync_copy(zeros_hbm.at[pl.ds(s * ZS, ZS)],
                        acc.at[pl.ds(s * ZS, ZS)])
        plsc.subcore_barrier()

        fast = c == 0
        nch = jnp.where(fast, NCH_F, NCH_S)
        base = pl.multiple_of(
            jnp.where(fast, OFF_F + s * EPW_F, s * EPW_S), CH)

        def src_chunk(j):
            return src_hbm.at[pl.ds(base + j * CH, CH)]

        def dst_chunk(j):
            return dst_hbm.at[pl.ds(base + j * CH, CH)]

        def slot(q):
            return rows.at[pl.ds(q * CH, CH)]

        for q in range(Q):
            pltpu.async_copy(src_chunk(q), sring.at[q], sem_is[q])
            pltpu.async_copy(dst_chunk(q), dring.at[q], sem_id[q])
        for q in range(Q):
            pltpu.make_async_copy(src_chunk(q), sring.at[q],
                                  sem_is[q]).wait()
            pltpu.async_copy(h_hbm.at[sring.at[q]], slot(q), sem_g[q])

        def body(m, carry):
            for q in range(Q):
                k = Q * m + q
                pltpu.make_async_copy(h_hbm.at[sring.at[q]], slot(q),
                                      sem_g[q]).wait()

                @pl.when(k + Q < nch)
                def _(q=q, k=k):
                    pltpu.async_copy(src_chunk(k + Q), sring.at[q],
                                     sem_is[q])

                pltpu.make_async_copy(dst_chunk(0), dring.at[q],
                                      sem_id[q]).wait()
                pltpu.sync_copy(slot(q), acc.at[dring.at[q]], add=True)

                @pl.when(k + Q < nch)
                def _(q=q, k=k):
                    pltpu.async_copy(dst_chunk(k + Q), dring.at[q],
                                     sem_id[q])
                    pltpu.make_async_copy(src_chunk(0), sring.at[q],
                                          sem_is[q]).wait()
                    pltpu.async_copy(h_hbm.at[sring.at[q]], slot(q),
                                     sem_g[q])

            return carry

        lax.fori_loop(0, nch // Q, body, 0)
        plsc.subcore_barrier()
        pltpu.sync_copy(acc.at[pl.ds(s * ZS, ZS)],
                        out_hbm.at[c, pl.ds(s * ZS, ZS)])

    return _sc_degree, _sc_edge_pass


def _tc_prep_body(x_ref, w_ref, d0_ref, d1_ref, dinv_ref, hp_ref):
    deg = d0_ref[...][:N, :1] + d1_ref[...][:N, :1] + 1.0
    dinv = lax.rsqrt(deg)
    dinv_ref[...] = dinv
    h = jnp.dot(x_ref[...], w_ref[...], preferred_element_type=jnp.float32)
    hp_ref[...] = h * dinv


def _tc_epi_body(relu, skip, wnext, refs):
    if skip and wnext:
        (s0, s1, hp, dinv, b, g, be, hprev, skw, w2, feat_ref, hn_ref) = refs
    elif wnext:
        (s0, s1, hp, dinv, b, g, be, w2, feat_ref, hn_ref) = refs
    else:
        (s0, s1, hp, dinv, b, g, be, feat_ref) = refs
    dv = dinv[...]
    conv = (s0[...][:N] + s1[...][:N] + hp[...]) * dv + b[...]
    m = jnp.mean(conv, axis=0, keepdims=True)
    v = jnp.mean((conv - m) ** 2, axis=0, keepdims=True)
    h = (conv - m) * lax.rsqrt(v + EPS) * g[...] + be[...]
    if relu:
        h = jnp.maximum(h, 0.0)
    if skip:
        h = h + skw[...] * hprev[...]
    feat_ref[...] = h
    if wnext:
        hn_ref[...] = jnp.dot(h, w2[...],
                              preferred_element_type=jnp.float32) * dv


_f = jnp.float32


def _tc_prep(x, w, d0, d1):
    return pl.pallas_call(
        _tc_prep_body,
        out_shape=(jax.ShapeDtypeStruct((N, 1), _f),
                   jax.ShapeDtypeStruct((N, D), _f)),
    )(x, w, d0, d1)


def _tc_epi(relu, skip, wnext, *args):
    outs = [jax.ShapeDtypeStruct((N, D), _f)]
    if wnext:
        outs.append(jax.ShapeDtypeStruct((N, D), _f))
    body = functools.partial(_tc_epi_body, relu, skip, wnext)
    return pl.pallas_call(
        lambda *refs: body(refs),
        out_shape=tuple(outs) if wnext else outs[0],
    )(*args)


def kernel(x, edge_index, W0, b0, g0, be0, W1, b1, g1, be1,
           W2, b2, g2, be2, W3, b3, g3, be3, skip_weight):
    src = edge_index[0].astype(jnp.int32)
    dst = edge_index[1].astype(jnp.int32)

    pad = E_PAD - E
    dst3 = jnp.concatenate([dst, jnp.full((pad,), DUMMY, jnp.int32)]
                           ).reshape(NW, NCH, CH)

    fper = (E - OFF_F) // NS
    fpad = EPW_F - fper

    def _split(a, fill):
        fast_part = a[OFF_F:].reshape(NS, fper)
        fast_part = jnp.pad(fast_part, ((0, 0), (0, fpad)),
                            constant_values=fill)
        return jnp.concatenate([a[:OFF_F], fast_part.reshape(-1)])

    src1 = _split(src, 0)
    dst1 = _split(dst, DUMMY)
    zeros_h = jnp.zeros((ACC_ROWS, D), _f)
    ones_h = jnp.ones((CH, D), _f)

    _sc_degree, _sc_edge_pass = _sc_kernels()
    dpar = _sc_degree(dst3, ones_h, zeros_h)
    dinv, hp = _tc_prep(x, W0, dpar[0], dpar[1])

    def edge(hprime):
        S = _sc_edge_pass(hprime, src1, dst1, zeros_h)
        return S[0], S[1]

    b0r, g0r, be0r = b0.reshape(1, D), g0.reshape(1, D), be0.reshape(1, D)
    b1r, g1r, be1r = b1.reshape(1, D), g1.reshape(1, D), be1.reshape(1, D)
    b2r, g2r, be2r = b2.reshape(1, D), g2.reshape(1, D), be2.reshape(1, D)
    b3r, g3r, be3r = b3.reshape(1, D), g3.reshape(1, D), be3.reshape(1, D)
    skw = jnp.asarray(skip_weight, _f).reshape(1, 1)

    s0, s1 = edge(hp)
    feat0, hp1 = _tc_epi(True, False, True, s0, s1, hp, dinv,
                         b0r, g0r, be0r, W1)
    s0, s1 = edge(hp1)
    feat1, hp2 = _tc_epi(True, True, True, s0, s1, hp1, dinv,
                         b1r, g1r, be1r, feat0, skw, W2)
    s0, s1 = edge(hp2)
    feat2, hp3 = _tc_epi(True, True, True, s0, s1, hp2, dinv,
                         b2r, g2r, be2r, feat1, skw, W3)
    s0, s1 = edge(hp3)
    out = _tc_epi(False, False, False, s0, s1, hp3, dinv, b3r, g3r, be3r)
    return out
